# baseline (speedup 1.0000x reference)
import functools

import jax
import jax.numpy as jnp
from jax import lax
from jax.experimental import pallas as pl
from jax.experimental.pallas import tpu as pltpu
from jax.experimental.pallas import tpu_sc as plsc

_NUM_CODES = 512
_CODE_DIM = 64


def _vq_body(x_ref, cba_ref, cbsq_ref, cb_ref, q_ref, loss_ref, idx_ref):
    b = pl.program_id(0)

    C = x_ref.shape[1]
    xb = x_ref[0].reshape(C, -1)
    cbm2 = cba_ref[...]
    cb_sq = cbsq_ref[...]

    dot = jax.lax.dot_general(
        cbm2, xb, (((1,), (0,)), ((), ())), preferred_element_type=jnp.float32
    )
    scores = dot + cb_sq
    idx = jnp.argmin(scores, axis=0)[None, :]
    idx_ref[0] = idx
    T = xb.shape[1]
    loc = jnp.broadcast_to(idx & 127, (C, T))
    chunk = jnp.broadcast_to(idx >> 7, (C, T))
    q = jnp.take_along_axis(cb_ref[:, 0:128], loc, axis=1)
    for g in range(1, _NUM_CODES // 128):
        qg = jnp.take_along_axis(cb_ref[:, g * 128:(g + 1) * 128], loc, axis=1)
        q = jnp.where(chunk == g, qg, q)
    q_ref[0] = q.reshape(q_ref.shape[1:])

    diff = xb - q
    part_loss = jnp.sum(diff * diff)

    @pl.when(b == 0)
    def _init():
        loss_ref[...] = jnp.zeros_like(loss_ref)

    loss_ref[...] += part_loss.reshape(1, 1)


def _tc_stage(x, codebook, interpret=False):
    B, C, H, W = x.shape
    cb_sq = jnp.sum(codebook * codebook, axis=1, keepdims=True)
    cbm2 = -2.0 * codebook
    grid = (B,)
    return pl.pallas_call(
        _vq_body,
        grid=grid,
        in_specs=[
            pl.BlockSpec((1, C, H, W), lambda b: (b, 0, 0, 0)),
            pl.BlockSpec((_NUM_CODES, _CODE_DIM), lambda b: (0, 0)),
            pl.BlockSpec((_NUM_CODES, 1), lambda b: (0, 0)),
            pl.BlockSpec((_CODE_DIM, _NUM_CODES), lambda b: (0, 0)),
        ],
        out_specs=[
            pl.BlockSpec((1, C, H, W), lambda b: (b, 0, 0, 0)),
            pl.BlockSpec((1, 1), lambda b: (0, 0)),
            pl.BlockSpec((1, 1, H * W), lambda b: (b, 0, 0)),
        ],
        out_shape=[
            jax.ShapeDtypeStruct((B, C, H, W), jnp.float32),
            jax.ShapeDtypeStruct((1, 1), jnp.float32),
            jax.ShapeDtypeStruct((B, 1, H * W), jnp.int32),
        ],
        interpret=interpret,
    )(x, cbm2, cb_sq, codebook.T)


def _make_sc_bincount(B, HW):
    info = plsc.get_sparse_core_info()
    NC, NS = info.num_cores, info.num_subcores
    NW = NC * NS
    assert B == NW, (B, NW)

    mesh = plsc.VectorSubcoreMesh(core_axis_name="c", subcore_axis_name="s")

    @functools.partial(
        pl.kernel,
        mesh=mesh,
        out_type=jax.ShapeDtypeStruct((NC, _NUM_CODES), jnp.float32),
        scratch_types=[
            pltpu.VMEM((HW,), jnp.int32),
            pltpu.VMEM((HW,), jnp.float32),
            pltpu.VMEM((_NUM_CODES,), jnp.float32),
            pltpu.VMEM_SHARED((_NUM_CODES,), jnp.float32),
        ],
    )
    def sc_bincount(idx_hbm, ones_hbm, zeros_hbm, out_hbm, idx_v, ones_v,
                    part_v, shared):
        cid = lax.axis_index("c")
        sid = lax.axis_index("s")
        wid = cid * NS + sid

        @pl.when(sid == 0)
        def _init():
            pltpu.sync_copy(zeros_hbm, part_v)
            pltpu.sync_copy(part_v, shared)

        pltpu.sync_copy(idx_hbm.at[wid, 0], idx_v)
        pltpu.sync_copy(ones_hbm, ones_v)
        plsc.subcore_barrier()
        pltpu.sync_copy(ones_v, shared.at[idx_v], add=True)
        plsc.subcore_barrier()

        @pl.when(sid == 0)
        def _drain():
            pltpu.sync_copy(shared, part_v)
            pltpu.sync_copy(part_v, out_hbm.at[cid])

    return sc_bincount


def _vq(x, codebook, interpret=False):
    B, C, H, W = x.shape
    q, loss_sum, idx = _tc_stage(x, codebook, interpret=interpret)
    if interpret:
        counts = jnp.sum(
            jax.nn.one_hot(idx[:, 0, :], _NUM_CODES, dtype=jnp.float32),
            axis=(0, 1),
        )
    else:
        ones = jnp.ones((H * W,), jnp.float32)
        zeros = jnp.zeros((_NUM_CODES,), jnp.float32)
        parts = _make_sc_bincount(B, H * W)(idx, ones, zeros)
        counts = jnp.sum(parts, axis=0)
    mse = loss_sum[0, 0] / x.size
    unique = jnp.sum(counts > 0.0)
    return q, mse, mse, unique


def kernel(x, codebook):
    return _vq(x, codebook)


# speedup vs baseline: 1.3645x; 1.3645x over previous
import functools

import jax
import jax.numpy as jnp
from jax import lax
from jax.experimental import pallas as pl
from jax.experimental.pallas import tpu as pltpu
from jax.experimental.pallas import tpu_sc as plsc

_NUM_CODES = 512
_CODE_DIM = 64


def _vq_body(x_ref, cba_ref, cbsq_ref, cb_ref, q_ref, loss_ref, idx_ref):
    b = pl.program_id(0)

    C = x_ref.shape[1]
    xb = x_ref[0].reshape(C, -1)
    cbm2 = cba_ref[...]
    cb_sq = cbsq_ref[...]

    dot = jax.lax.dot_general(
        cbm2, xb, (((1,), (0,)), ((), ())), preferred_element_type=jnp.float32
    )
    scores = dot + cb_sq
    idx = jnp.argmin(scores, axis=0)[None, :]
    idx_ref[0] = idx
    iota = jax.lax.broadcasted_iota(jnp.int32, scores.shape, 0)
    onehot = (iota == idx).astype(jnp.float32)
    q = jax.lax.dot_general(
        cb_ref[...], onehot, (((0,), (0,)), ((), ())),
        preferred_element_type=jnp.float32,
    )
    q_ref[0] = q.reshape(q_ref.shape[1:])

    diff = xb - q
    part_loss = jnp.sum(diff * diff)

    @pl.when(b == 0)
    def _init():
        loss_ref[...] = jnp.zeros_like(loss_ref)

    loss_ref[...] += part_loss.reshape(1, 1)


def _tc_stage(x, codebook, interpret=False):
    B, C, H, W = x.shape
    cb_sq = jnp.sum(codebook * codebook, axis=1, keepdims=True)
    cbm2 = -2.0 * codebook
    grid = (B,)
    return pl.pallas_call(
        _vq_body,
        grid=grid,
        in_specs=[
            pl.BlockSpec((1, C, H, W), lambda b: (b, 0, 0, 0)),
            pl.BlockSpec((_NUM_CODES, _CODE_DIM), lambda b: (0, 0)),
            pl.BlockSpec((_NUM_CODES, 1), lambda b: (0, 0)),
            pl.BlockSpec((_NUM_CODES, _CODE_DIM), lambda b: (0, 0)),
        ],
        out_specs=[
            pl.BlockSpec((1, C, H, W), lambda b: (b, 0, 0, 0)),
            pl.BlockSpec((1, 1), lambda b: (0, 0)),
            pl.BlockSpec((1, 1, H * W), lambda b: (b, 0, 0)),
        ],
        out_shape=[
            jax.ShapeDtypeStruct((B, C, H, W), jnp.float32),
            jax.ShapeDtypeStruct((1, 1), jnp.float32),
            jax.ShapeDtypeStruct((B, 1, H * W), jnp.int32),
        ],
        interpret=interpret,
    )(x, cbm2, cb_sq, codebook)


def _make_sc_bincount(B, HW):
    info = plsc.get_sparse_core_info()
    NC, NS = info.num_cores, info.num_subcores
    NW = NC * NS
    assert B == NW, (B, NW)

    mesh = plsc.VectorSubcoreMesh(core_axis_name="c", subcore_axis_name="s")

    @functools.partial(
        pl.kernel,
        mesh=mesh,
        out_type=jax.ShapeDtypeStruct((NC, _NUM_CODES), jnp.float32),
        scratch_types=[
            pltpu.VMEM((HW,), jnp.int32),
            pltpu.VMEM((HW,), jnp.float32),
            pltpu.VMEM((_NUM_CODES,), jnp.float32),
            pltpu.VMEM_SHARED((_NUM_CODES,), jnp.float32),
        ],
    )
    def sc_bincount(idx_hbm, ones_hbm, zeros_hbm, out_hbm, idx_v, ones_v,
                    part_v, shared):
        cid = lax.axis_index("c")
        sid = lax.axis_index("s")
        wid = cid * NS + sid

        @pl.when(sid == 0)
        def _init():
            pltpu.sync_copy(zeros_hbm, part_v)
            pltpu.sync_copy(part_v, shared)

        pltpu.sync_copy(idx_hbm.at[wid, 0], idx_v)
        pltpu.sync_copy(ones_hbm, ones_v)
        plsc.subcore_barrier()
        pltpu.sync_copy(ones_v, shared.at[idx_v], add=True)
        plsc.subcore_barrier()

        @pl.when(sid == 0)
        def _drain():
            pltpu.sync_copy(shared, part_v)
            pltpu.sync_copy(part_v, out_hbm.at[cid])

    return sc_bincount


def _vq(x, codebook, interpret=False):
    B, C, H, W = x.shape
    q, loss_sum, idx = _tc_stage(x, codebook, interpret=interpret)
    if interpret:
        counts = jnp.sum(
            jax.nn.one_hot(idx[:, 0, :], _NUM_CODES, dtype=jnp.float32),
            axis=(0, 1),
        )
    else:
        ones = jnp.ones((H * W,), jnp.float32)
        zeros = jnp.zeros((_NUM_CODES,), jnp.float32)
        parts = _make_sc_bincount(B, H * W)(idx, ones, zeros)
        counts = jnp.sum(parts, axis=0)
    mse = loss_sum[0, 0] / x.size
    unique = jnp.sum(counts > 0.0)
    return q, mse, mse, unique


def kernel(x, codebook):
    return _vq(x, codebook)
